# baseline (speedup 1.0000x reference)
import jax
import jax.numpy as jnp
from jax import lax
from jax.experimental import pallas as pl
from jax.experimental.pallas import tpu as pltpu
from jax.experimental.pallas import tpu_sc as plsc

N = 10000
NP = 10240
E = 320000
D_IN = 128
H = 64
GP = 16

R = 1024
CH = 128
NCHUNK = E // CH
NS = 16
ROWS_PER_TILE = NP // NS

P1_PER_TILE = NCHUNK // NS
BLK = 12
P1_NBLK = P1_PER_TILE // BLK
P2_PER_CORE = NCHUNK // 2
P2_PER_TILE = P2_PER_CORE // NS
P2_NBLK = P2_PER_TILE // BLK
P2_TAIL = P2_PER_TILE - P2_NBLK * BLK
NB = 4

_F32 = jnp.float32
_HI = jax.lax.Precision.HIGHEST
_SC_PARAMS = pltpu.CompilerParams(use_tc_tiling_on_sc=False,
                                  needs_layout_passes=False)


def _enc_body(xu, xm, Wu, bu, Wm, bm, W1lum, W1rmu, W1lmu, W1rum,
              A, B, Hrm, Hru):
    hu = jnp.dot(xu[...], Wu[...], precision=_HI, preferred_element_type=_F32) + bu[...]
    hm = jnp.dot(xm[...], Wm[...], precision=_HI, preferred_element_type=_F32) + bm[...]
    A[...] = jnp.dot(hu, W1lum[...], precision=_HI, preferred_element_type=_F32)
    Hru[...] = jnp.dot(hu, W1rmu[...], precision=_HI, preferred_element_type=_F32)
    B[...] = jnp.dot(hm, W1lmu[...], precision=_HI, preferred_element_type=_F32)
    Hrm[...] = jnp.dot(hm, W1rum[...], precision=_HI, preferred_element_type=_F32)


def _encode(xu, xm, Wu, bu, Wm, bm, W1lum, W1rmu, W1lmu, W1rum):
    blk_x = pl.BlockSpec((R, D_IN), lambda i: (i, 0))
    blk_w = pl.BlockSpec((D_IN, H), lambda i: (0, 0))
    blk_b = pl.BlockSpec((1, H), lambda i: (0, 0))
    blk_w2 = pl.BlockSpec((H, H), lambda i: (0, 0))
    blk_o = pl.BlockSpec((R, H), lambda i: (i, 0))
    out = jax.ShapeDtypeStruct((NP, H), _F32)
    return pl.pallas_call(
        _enc_body,
        grid=(NP // R,),
        in_specs=[blk_x, blk_x, blk_w, blk_b, blk_w, blk_b,
                  blk_w2, blk_w2, blk_w2, blk_w2],
        out_specs=[blk_o, blk_o, blk_o, blk_o],
        out_shape=[out, out, out, out],
    )(xu, xm, Wu, bu, Wm, bm, W1lum, W1rmu, W1lmu, W1rum)


def _mid_body(Sm, cm, Hrm, b1um, Su, cu, Hru, b1mu, W2l, W2r, b2l,
              t_out, base_out):
    m1 = jnp.maximum(Sm[...] / jnp.clip(cm[...], 1.0, None)
                     + b1um[...] + Hrm[...], 0.0)
    u1 = jnp.maximum(Su[...] / jnp.clip(cu[...], 1.0, None)
                     + b1mu[...] + Hru[...], 0.0)
    t_out[...] = jnp.dot(u1, W2l[...], precision=_HI, preferred_element_type=_F32)
    base_out[...] = jnp.dot(m1, W2r[...], precision=_HI,
                            preferred_element_type=_F32) + b2l[...]


def _mid(Sm, cm, Hrm, b1um, Su, cu, Hru, b1mu, W2l, W2r, b2l):
    blk_s = pl.BlockSpec((R, H), lambda i: (i, 0))
    blk_c = pl.BlockSpec((R, 1), lambda i: (i, 0))
    blk_b = pl.BlockSpec((1, H), lambda i: (0, 0))
    blk_w = pl.BlockSpec((H, GP), lambda i: (0, 0))
    blk_b2 = pl.BlockSpec((1, GP), lambda i: (0, 0))
    blk_o = pl.BlockSpec((R, GP), lambda i: (i, 0))
    out = jax.ShapeDtypeStruct((NP, GP), _F32)
    return pl.pallas_call(
        _mid_body,
        grid=(NP // R,),
        in_specs=[blk_s, blk_c, blk_s, blk_b, blk_s, blk_c, blk_s, blk_b,
                  blk_w, blk_w, blk_b2],
        out_specs=[blk_o, blk_o],
        out_shape=[out, out],
    )(Sm, cm, Hrm, b1um, Su, cu, Hru, b1mu, W2l, W2r, b2l)


def _final_body(S2a, S2b, cm, base, out):
    out[...] = (S2a[...] + S2b[...]) / jnp.clip(cm[...], 1.0, None) + base[...]


def _final(S2a, S2b, cm, base):
    blk = pl.BlockSpec((R, GP), lambda i: (i, 0))
    blk_c = pl.BlockSpec((R, 1), lambda i: (i, 0))
    return pl.pallas_call(
        _final_body,
        grid=(NP // R,),
        in_specs=[blk, blk, blk_c, blk],
        out_specs=blk,
        out_shape=jax.ShapeDtypeStruct((NP, GP), _F32),
    )(S2a, S2b, cm, base)


def _fill_zeros(ref, nrows, ncols):
    @pl.loop(0, nrows)
    def _(i):
        @pl.loop(0, ncols, step=GP)
        def _(j):
            ref.at[i, pl.ds(j, GP)][...] = jnp.zeros((GP,), _F32)


def _sparse1_body(A_hbm, B_hbm, idx2_hbm,
                  Sm_hbm, cm_hbm, Su_hbm, cu_hbm,
                  iba, ibb, rows, cnt_v, cbuf, zb64,
                  gsem, rsem, S_sh, c_all):
    c = lax.axis_index("c")
    t = lax.axis_index("s")
    base = t * ROWS_PER_TILE
    c0 = t * P1_PER_TILE
    ones16 = jnp.ones((16,), _F32)

    _fill_zeros(zb64, CH, H)

    @pl.loop(0, NP // 16)
    def _(i):
        cnt_v.at[pl.ds(i * 16, 16)][...] = jnp.zeros((16,), _F32)

    @pl.loop(0, ROWS_PER_TILE // CH)
    def _(k):
        pltpu.sync_copy(zb64, S_sh.at[pl.ds(base + k * CH, CH)])

    plsc.subcore_barrier()

    def run_dir(tab_hbm, gslot, sslot):
        def fire_g(ibuf, k, b):
            pltpu.async_copy(tab_hbm.at[ibuf.at[k, gslot]], rows.at[b],
                             gsem.at[b])

        def wait_g(b):
            pltpu.make_async_copy(tab_hbm.at[iba.at[0, gslot]], rows.at[b],
                                  gsem.at[b]).wait()

        def fire_s(ibuf, k, b):
            pltpu.async_copy(rows.at[b], S_sh.at[ibuf.at[k, sslot]],
                             rsem.at[b], add=True)
            for r in range(CH // 16):
                ireg = ibuf.at[k, sslot, pl.ds(16 * r, 16)][...]
                plsc.addupdate_scatter(cnt_v, [ireg], ones16)

        def wait_s(b):
            pltpu.make_async_copy(rows.at[b], S_sh.at[iba.at[0, sslot]],
                                  rsem.at[b]).wait()

        def do_block(ibuf, notfirst):
            def guarded(f):
                if notfirst is True:
                    f()
                else:
                    pl.when(notfirst)(f)

            guarded(lambda: wait_s(0))
            fire_g(ibuf, 0, 0)
            for j in range(BLK):
                b = j % NB
                if j < BLK - 1:
                    bn = (j + 1) % NB
                    if j < NB - 1:
                        guarded(lambda bn=bn: wait_s(bn))
                    else:
                        wait_s(bn)
                    fire_g(ibuf, j + 1, bn)
                wait_g(b)
                fire_s(ibuf, j, b)

        def load_blk(ibuf, bi):
            pltpu.sync_copy(idx2_hbm.at[pl.ds(c0 + bi * BLK, BLK)], ibuf)

        load_blk(iba, 0)

        @pl.loop(0, P1_NBLK // 2)
        def _(p):
            do_block(iba, p > 0)
            load_blk(ibb, 2 * p + 1)
            do_block(ibb, True)
            load_blk(iba, 2 * p + 2)

        do_block(iba, True)

        for b in range(NB):
            wait_s(b)

        @pl.when(t < NCHUNK - NS * P1_PER_TILE)
        def _():
            pltpu.sync_copy(idx2_hbm.at[pl.ds(NS * P1_PER_TILE + t, 1)],
                            iba.at[pl.ds(0, 1)])
            pltpu.sync_copy(tab_hbm.at[iba.at[0, gslot]], rows.at[0])
            pltpu.sync_copy(rows.at[0], S_sh.at[iba.at[0, sslot]], add=True)
            for r in range(CH // 16):
                ireg = iba.at[0, sslot, pl.ds(16 * r, 16)][...]
                plsc.addupdate_scatter(cnt_v, [ireg], ones16)

    @pl.when(c == 0)
    def _():
        run_dir(A_hbm, 0, 1)

    @pl.when(c == 1)
    def _():
        run_dir(B_hbm, 1, 0)

    pltpu.sync_copy(cnt_v, c_all.at[t])
    plsc.subcore_barrier()
    pltpu.sync_copy(c_all.at[:, pl.ds(base, ROWS_PER_TILE)], cbuf)

    @pl.loop(0, ROWS_PER_TILE // 16)
    def _(g):
        acc = cbuf.at[0, pl.ds(g * 16, 16)][...]
        for r in range(1, NS):
            acc = acc + cbuf.at[r, pl.ds(g * 16, 16)][...]
        cbuf.at[0, pl.ds(g * 16, 16)][...] = acc

    @pl.when(c == 0)
    def _():
        pltpu.sync_copy(S_sh.at[pl.ds(base, ROWS_PER_TILE)],
                        Sm_hbm.at[pl.ds(base, ROWS_PER_TILE)])
        pltpu.sync_copy(cbuf.at[0], cm_hbm.at[pl.ds(base, ROWS_PER_TILE)])

    @pl.when(c == 1)
    def _():
        pltpu.sync_copy(S_sh.at[pl.ds(base, ROWS_PER_TILE)],
                        Su_hbm.at[pl.ds(base, ROWS_PER_TILE)])
        pltpu.sync_copy(cbuf.at[0], cu_hbm.at[pl.ds(base, ROWS_PER_TILE)])


def _sparse1(A, B, idx2):
    mesh = plsc.VectorSubcoreMesh(core_axis_name="c", subcore_axis_name="s")
    out = [jax.ShapeDtypeStruct((NP, H), _F32),
           jax.ShapeDtypeStruct((NP,), _F32),
           jax.ShapeDtypeStruct((NP, H), _F32),
           jax.ShapeDtypeStruct((NP,), _F32)]
    f = pl.kernel(
        _sparse1_body,
        out_type=out,
        mesh=mesh,
        compiler_params=_SC_PARAMS,
        scratch_types=[
            pltpu.VMEM((BLK, 2, CH), jnp.int32),
            pltpu.VMEM((BLK, 2, CH), jnp.int32),
            pltpu.VMEM((NB, CH, H), _F32),
            pltpu.VMEM((NP,), _F32),
            pltpu.VMEM((NS, ROWS_PER_TILE), _F32),
            pltpu.VMEM((CH, H), _F32),
            pltpu.SemaphoreType.DMA((NB,)),
            pltpu.SemaphoreType.DMA((NB,)),
            pltpu.VMEM_SHARED((NP, H), _F32),
            pltpu.VMEM_SHARED((NS, NP), _F32),
        ],
    )
    return f(A, B, idx2)


def _sparse2_body(T_hbm, idx2_hbm, S2_hbm,
                  iba, ibb, rows, zb16, gsem, rsem, tab_sh, S_sh):
    c = lax.axis_index("c")
    t = lax.axis_index("s")
    base = t * ROWS_PER_TILE
    c0 = c * P2_PER_CORE + t * P2_PER_TILE
    _fill_zeros(zb16, CH, GP)

    pltpu.sync_copy(T_hbm.at[pl.ds(base, ROWS_PER_TILE)],
                    tab_sh.at[pl.ds(base, ROWS_PER_TILE)])

    @pl.loop(0, ROWS_PER_TILE // CH)
    def _(k):
        pltpu.sync_copy(zb16, S_sh.at[pl.ds(base + k * CH, CH)])

    plsc.subcore_barrier()

    def fire_g(ibuf, k, b):
        pltpu.async_copy(tab_sh.at[ibuf.at[k, 0]], rows.at[b], gsem.at[b])

    def wait_g(b):
        pltpu.make_async_copy(tab_sh.at[iba.at[0, 0]], rows.at[b],
                              gsem.at[b]).wait()

    def fire_s(ibuf, k, b):
        pltpu.async_copy(rows.at[b], S_sh.at[ibuf.at[k, 1]], rsem.at[b],
                         add=True)

    def wait_s(b):
        pltpu.make_async_copy(rows.at[b], S_sh.at[iba.at[0, 1]],
                              rsem.at[b]).wait()

    def do_block(ibuf, notfirst):
        def guarded(f):
            if notfirst is True:
                f()
            else:
                pl.when(notfirst)(f)

        guarded(lambda: wait_s(0))
        fire_g(ibuf, 0, 0)
        for j in range(BLK):
            b = j % NB
            if j < BLK - 1:
                bn = (j + 1) % NB
                if j < NB - 1:
                    guarded(lambda bn=bn: wait_s(bn))
                else:
                    wait_s(bn)
                fire_g(ibuf, j + 1, bn)
            wait_g(b)
            fire_s(ibuf, j, b)

    def load_blk(ibuf, bi):
        pltpu.sync_copy(idx2_hbm.at[pl.ds(c0 + bi * BLK, BLK)], ibuf)

    load_blk(iba, 0)

    @pl.loop(0, P2_NBLK // 2)
    def _(p):
        do_block(iba, p > 0)
        load_blk(ibb, 2 * p + 1)
        do_block(ibb, True)

        @pl.when(p < P2_NBLK // 2 - 1)
        def _():
            load_blk(iba, 2 * p + 2)

    for b in range(NB):
        wait_s(b)

    @pl.loop(0, P2_TAIL)
    def _(k):
        pltpu.sync_copy(idx2_hbm.at[pl.ds(c0 + P2_NBLK * BLK + k, 1)],
                        iba.at[pl.ds(0, 1)])
        pltpu.sync_copy(tab_sh.at[iba.at[0, 0]], rows.at[0])
        pltpu.sync_copy(rows.at[0], S_sh.at[iba.at[0, 1]], add=True)

    @pl.when(t < P2_PER_CORE - NS * P2_PER_TILE)
    def _():
        pltpu.sync_copy(
            idx2_hbm.at[pl.ds(c * P2_PER_CORE + NS * P2_PER_TILE + t, 1)],
            iba.at[pl.ds(0, 1)])
        pltpu.sync_copy(tab_sh.at[iba.at[0, 0]], rows.at[0])
        pltpu.sync_copy(rows.at[0], S_sh.at[iba.at[0, 1]], add=True)

    plsc.subcore_barrier()
    pltpu.sync_copy(S_sh.at[pl.ds(base, ROWS_PER_TILE)],
                    S2_hbm.at[c, pl.ds(base, ROWS_PER_TILE)])


def _sparse2(T, idx2):
    mesh = plsc.VectorSubcoreMesh(core_axis_name="c", subcore_axis_name="s")
    f = pl.kernel(
        _sparse2_body,
        out_type=jax.ShapeDtypeStruct((2, NP, GP), _F32),
        mesh=mesh,
        compiler_params=_SC_PARAMS,
        scratch_types=[
            pltpu.VMEM((BLK, 2, CH), jnp.int32),
            pltpu.VMEM((BLK, 2, CH), jnp.int32),
            pltpu.VMEM((NB, CH, GP), _F32),
            pltpu.VMEM((CH, GP), _F32),
            pltpu.SemaphoreType.DMA((NB,)),
            pltpu.SemaphoreType.DMA((NB,)),
            pltpu.VMEM_SHARED((NP, GP), _F32),
            pltpu.VMEM_SHARED((NP, GP), _F32),
        ],
    )
    return f(T, idx2)


def kernel(x_user, x_movie, edge_index_rates, edge_index_rev,
           W_user, b_user, W_movie, b_movie,
           W1l_um, b1l_um, W1r_um, W1l_mu, b1l_mu, W1r_mu,
           W2l_um, b2l_um, W2r_um, W2l_mu, b2l_mu, W2r_mu):
    idx2 = jnp.stack([edge_index_rates[0].astype(jnp.int32).reshape(NCHUNK, CH),
                      edge_index_rates[1].astype(jnp.int32).reshape(NCHUNK, CH)],
                     axis=1)

    bu = b_user.reshape(1, H)
    bm = b_movie.reshape(1, H)
    b1um = b1l_um.reshape(1, H)
    b1mu = b1l_mu.reshape(1, H)
    W2l = jnp.zeros((H, GP), _F32).at[:, :3].set(W2l_um)
    W2r = jnp.zeros((H, GP), _F32).at[:, :3].set(W2r_um)
    b2l = jnp.zeros((1, GP), _F32).at[0, :3].set(b2l_um)

    A, B, Hrm, Hru = _encode(x_user, x_movie, W_user, bu, W_movie, bm,
                             W1l_um, W1r_mu, W1l_mu, W1r_um)
    Sm, cm, Su, cu = _sparse1(A, B, idx2)
    cm = cm.reshape(NP, 1)
    cu = cu.reshape(NP, 1)
    T, base = _mid(Sm, cm, Hrm, b1um, Su, cu, Hru, b1mu, W2l, W2r, b2l)
    S2 = _sparse2(T, idx2)
    out16 = _final(S2[0], S2[1], cm, base)
    return out16[:N, :3]


# speedup vs baseline: 1.0608x; 1.0608x over previous
import jax
import jax.numpy as jnp
from jax import lax
from jax.experimental import pallas as pl
from jax.experimental.pallas import tpu as pltpu
from jax.experimental.pallas import tpu_sc as plsc

N = 10000
NP = 10240
E = 320000
D_IN = 128
H = 64
GP = 16

R = 1024
CH = 128
NCHUNK = E // CH
NS = 16
ROWS_PER_TILE = NP // NS

P1_PER_TILE = NCHUNK // NS
BLK = 12
P1_NBLK = P1_PER_TILE // BLK
P2_PER_CORE = NCHUNK // 2
P2_PER_TILE = P2_PER_CORE // NS
P2_NBLK = P2_PER_TILE // BLK
P2_TAIL = P2_PER_TILE - P2_NBLK * BLK
NB = 4

_F32 = jnp.float32
_HI = jax.lax.Precision.HIGHEST
_SC_PARAMS = pltpu.CompilerParams(use_tc_tiling_on_sc=False,
                                  needs_layout_passes=False)


def _wcomb_body(Wu, bu, Wm, bm, W1lum, W1rmu, W1lmu, W1rum,
                Wu128, bu128, Wm128, bm128):
    Wu128[:, 0:H] = jnp.dot(Wu[...], W1lum[...], precision=_HI,
                            preferred_element_type=_F32)
    Wu128[:, H:2 * H] = jnp.dot(Wu[...], W1rmu[...], precision=_HI,
                                preferred_element_type=_F32)
    bu128[:, 0:H] = jnp.dot(bu[...], W1lum[...], precision=_HI,
                            preferred_element_type=_F32)
    bu128[:, H:2 * H] = jnp.dot(bu[...], W1rmu[...], precision=_HI,
                                preferred_element_type=_F32)
    Wm128[:, 0:H] = jnp.dot(Wm[...], W1lmu[...], precision=_HI,
                            preferred_element_type=_F32)
    Wm128[:, H:2 * H] = jnp.dot(Wm[...], W1rum[...], precision=_HI,
                                preferred_element_type=_F32)
    bm128[:, 0:H] = jnp.dot(bm[...], W1lmu[...], precision=_HI,
                            preferred_element_type=_F32)
    bm128[:, H:2 * H] = jnp.dot(bm[...], W1rum[...], precision=_HI,
                                preferred_element_type=_F32)


def _wcomb(Wu, bu, Wm, bm, W1lum, W1rmu, W1lmu, W1rum):
    return pl.pallas_call(
        _wcomb_body,
        out_shape=[jax.ShapeDtypeStruct((D_IN, 2 * H), _F32),
                   jax.ShapeDtypeStruct((1, 2 * H), _F32),
                   jax.ShapeDtypeStruct((D_IN, 2 * H), _F32),
                   jax.ShapeDtypeStruct((1, 2 * H), _F32)],
    )(Wu, bu, Wm, bm, W1lum, W1rmu, W1lmu, W1rum)


def _enc_body(xu, xm, Wu128, bu128, Wm128, bm128, U, M, A, B):
    u = jnp.dot(xu[...], Wu128[...], precision=_HI,
                preferred_element_type=_F32) + bu128[...]
    m = jnp.dot(xm[...], Wm128[...], precision=_HI,
                preferred_element_type=_F32) + bm128[...]
    U[...] = u
    M[...] = m
    A[...] = u[:, 0:H]
    B[...] = m[:, 0:H]


def _encode(xu, xm, Wu128, bu128, Wm128, bm128):
    blk_x = pl.BlockSpec((R, D_IN), lambda i: (i, 0))
    blk_w = pl.BlockSpec((D_IN, 2 * H), lambda i: (0, 0))
    blk_b = pl.BlockSpec((1, 2 * H), lambda i: (0, 0))
    blk_o = pl.BlockSpec((R, 2 * H), lambda i: (i, 0))
    blk_h = pl.BlockSpec((R, H), lambda i: (i, 0))
    out = jax.ShapeDtypeStruct((NP, 2 * H), _F32)
    outh = jax.ShapeDtypeStruct((NP, H), _F32)
    return pl.pallas_call(
        _enc_body,
        grid=(NP // R,),
        in_specs=[blk_x, blk_x, blk_w, blk_b, blk_w, blk_b],
        out_specs=[blk_o, blk_o, blk_h, blk_h],
        out_shape=[out, out, outh, outh],
    )(xu, xm, Wu128, bu128, Wm128, bm128)


def _mid_body(SS, cm, cu, U, M, b1um, b1mu, W2l, W2r, b2l,
              t_out, base_out):
    m1 = jnp.maximum(SS[...][:, 0:H] / jnp.clip(cm[...], 1.0, None)
                     + b1um[...] + M[...][:, H:2 * H], 0.0)
    u1 = jnp.maximum(SS[...][:, H:2 * H] / jnp.clip(cu[...], 1.0, None)
                     + b1mu[...] + U[...][:, H:2 * H], 0.0)
    t_out[...] = jnp.dot(u1, W2l[...], precision=_HI, preferred_element_type=_F32)
    base_out[...] = jnp.dot(m1, W2r[...], precision=_HI,
                            preferred_element_type=_F32) + b2l[...]


def _mid(SS, cm, cu, U, M, b1um, b1mu, W2l, W2r, b2l):
    blk_s = pl.BlockSpec((R, 2 * H), lambda i: (i, 0))
    blk_c = pl.BlockSpec((R, 1), lambda i: (i, 0))
    blk_b = pl.BlockSpec((1, H), lambda i: (0, 0))
    blk_w = pl.BlockSpec((H, GP), lambda i: (0, 0))
    blk_b2 = pl.BlockSpec((1, GP), lambda i: (0, 0))
    blk_o = pl.BlockSpec((R, GP), lambda i: (i, 0))
    out = jax.ShapeDtypeStruct((NP, GP), _F32)
    return pl.pallas_call(
        _mid_body,
        grid=(NP // R,),
        in_specs=[blk_s, blk_c, blk_c, blk_s, blk_s, blk_b, blk_b,
                  blk_w, blk_w, blk_b2],
        out_specs=[blk_o, blk_o],
        out_shape=[out, out],
    )(SS, cm, cu, U, M, b1um, b1mu, W2l, W2r, b2l)


def _final_body(S2a, S2b, cm, base, out):
    out[...] = (S2a[...] + S2b[...]) / jnp.clip(cm[...], 1.0, None) + base[...]


def _final(S2a, S2b, cm, base):
    blk = pl.BlockSpec((R, GP), lambda i: (i, 0))
    blk_c = pl.BlockSpec((R, 1), lambda i: (i, 0))
    return pl.pallas_call(
        _final_body,
        grid=(NP // R,),
        in_specs=[blk, blk, blk_c, blk],
        out_specs=blk,
        out_shape=jax.ShapeDtypeStruct((NP, GP), _F32),
    )(S2a, S2b, cm, base)


def _fill_zeros(ref, nrows, ncols):
    @pl.loop(0, nrows)
    def _(i):
        @pl.loop(0, ncols, step=GP)
        def _(j):
            ref.at[i, pl.ds(j, GP)][...] = jnp.zeros((GP,), _F32)


def _sparse1_body(A_hbm, B_hbm, idx2_hbm,
                  SS_hbm, cm_hbm, cu_hbm,
                  iba, ibb, rows, cnt_v, cbuf, zb64,
                  gsem, rsem, S_sh, c_all):
    c = lax.axis_index("c")
    t = lax.axis_index("s")
    base = t * ROWS_PER_TILE
    c0 = t * P1_PER_TILE
    ones16 = jnp.ones((16,), _F32)

    _fill_zeros(zb64, CH, H)

    @pl.loop(0, NP // 16)
    def _(i):
        cnt_v.at[pl.ds(i * 16, 16)][...] = jnp.zeros((16,), _F32)

    @pl.loop(0, ROWS_PER_TILE // CH)
    def _(k):
        pltpu.sync_copy(zb64, S_sh.at[pl.ds(base + k * CH, CH)])

    plsc.subcore_barrier()

    def run_dir(tab_hbm, gslot, sslot):
        def fire_g(ibuf, k, b):
            pltpu.async_copy(tab_hbm.at[ibuf.at[k, gslot]], rows.at[b],
                             gsem.at[b])

        def wait_g(b):
            pltpu.make_async_copy(tab_hbm.at[iba.at[0, gslot]], rows.at[b],
                                  gsem.at[b]).wait()

        def fire_s(ibuf, k, b):
            pltpu.async_copy(rows.at[b], S_sh.at[ibuf.at[k, sslot]],
                             rsem.at[b], add=True)
            for r in range(CH // 16):
                ireg = ibuf.at[k, sslot, pl.ds(16 * r, 16)][...]
                plsc.addupdate_scatter(cnt_v, [ireg], ones16)

        def wait_s(b):
            pltpu.make_async_copy(rows.at[b], S_sh.at[iba.at[0, sslot]],
                                  rsem.at[b]).wait()

        def do_block(ibuf, notfirst):
            def guarded(f):
                if notfirst is True:
                    f()
                else:
                    pl.when(notfirst)(f)

            guarded(lambda: wait_s(0))
            fire_g(ibuf, 0, 0)
            for j in range(BLK):
                b = j % NB
                if j < BLK - 1:
                    bn = (j + 1) % NB
                    if j < NB - 1:
                        guarded(lambda bn=bn: wait_s(bn))
                    else:
                        wait_s(bn)
                    fire_g(ibuf, j + 1, bn)
                wait_g(b)
                fire_s(ibuf, j, b)

        def load_blk(ibuf, bi):
            pltpu.sync_copy(idx2_hbm.at[pl.ds(c0 + bi * BLK, BLK)], ibuf)

        load_blk(iba, 0)

        @pl.loop(0, P1_NBLK // 2)
        def _(p):
            do_block(iba, p > 0)
            load_blk(ibb, 2 * p + 1)
            do_block(ibb, True)
            load_blk(iba, 2 * p + 2)

        do_block(iba, True)

        for b in range(NB):
            wait_s(b)

        @pl.when(t < NCHUNK - NS * P1_PER_TILE)
        def _():
            pltpu.sync_copy(idx2_hbm.at[pl.ds(NS * P1_PER_TILE + t, 1)],
                            iba.at[pl.ds(0, 1)])
            pltpu.sync_copy(tab_hbm.at[iba.at[0, gslot]], rows.at[0])
            pltpu.sync_copy(rows.at[0], S_sh.at[iba.at[0, sslot]], add=True)
            for r in range(CH // 16):
                ireg = iba.at[0, sslot, pl.ds(16 * r, 16)][...]
                plsc.addupdate_scatter(cnt_v, [ireg], ones16)

    @pl.when(c == 0)
    def _():
        run_dir(A_hbm, 0, 1)

    @pl.when(c == 1)
    def _():
        run_dir(B_hbm, 1, 0)

    pltpu.sync_copy(cnt_v, c_all.at[t])
    plsc.subcore_barrier()
    pltpu.sync_copy(c_all.at[:, pl.ds(base, ROWS_PER_TILE)], cbuf)

    @pl.loop(0, ROWS_PER_TILE // 16)
    def _(g):
        acc = cbuf.at[0, pl.ds(g * 16, 16)][...]
        for r in range(1, NS):
            acc = acc + cbuf.at[r, pl.ds(g * 16, 16)][...]
        cbuf.at[0, pl.ds(g * 16, 16)][...] = acc

    pltpu.sync_copy(S_sh.at[pl.ds(base, ROWS_PER_TILE)],
                    SS_hbm.at[pl.ds(base, ROWS_PER_TILE), pl.ds(c * H, H)])

    @pl.when(c == 0)
    def _():
        pltpu.sync_copy(cbuf.at[0], cm_hbm.at[pl.ds(base, ROWS_PER_TILE)])

    @pl.when(c == 1)
    def _():
        pltpu.sync_copy(cbuf.at[0], cu_hbm.at[pl.ds(base, ROWS_PER_TILE)])


def _sparse1(A, B, idx2):
    mesh = plsc.VectorSubcoreMesh(core_axis_name="c", subcore_axis_name="s")
    out = [jax.ShapeDtypeStruct((NP, 2 * H), _F32),
           jax.ShapeDtypeStruct((NP,), _F32),
           jax.ShapeDtypeStruct((NP,), _F32)]
    f = pl.kernel(
        _sparse1_body,
        out_type=out,
        mesh=mesh,
        compiler_params=_SC_PARAMS,
        scratch_types=[
            pltpu.VMEM((BLK, 2, CH), jnp.int32),
            pltpu.VMEM((BLK, 2, CH), jnp.int32),
            pltpu.VMEM((NB, CH, H), _F32),
            pltpu.VMEM((NP,), _F32),
            pltpu.VMEM((NS, ROWS_PER_TILE), _F32),
            pltpu.VMEM((CH, H), _F32),
            pltpu.SemaphoreType.DMA((NB,)),
            pltpu.SemaphoreType.DMA((NB,)),
            pltpu.VMEM_SHARED((NP, H), _F32),
            pltpu.VMEM_SHARED((NS, NP), _F32),
        ],
    )
    return f(A, B, idx2)


def _sparse2_body(T_hbm, idx2_hbm, S2_hbm,
                  iba, ibb, rows, zb16, gsem, rsem, tab_sh, S_sh):
    c = lax.axis_index("c")
    t = lax.axis_index("s")
    base = t * ROWS_PER_TILE
    c0 = c * P2_PER_CORE + t * P2_PER_TILE
    _fill_zeros(zb16, CH, GP)

    pltpu.sync_copy(T_hbm.at[pl.ds(base, ROWS_PER_TILE)],
                    tab_sh.at[pl.ds(base, ROWS_PER_TILE)])

    @pl.loop(0, ROWS_PER_TILE // CH)
    def _(k):
        pltpu.sync_copy(zb16, S_sh.at[pl.ds(base + k * CH, CH)])

    plsc.subcore_barrier()

    def fire_g(ibuf, k, b):
        pltpu.async_copy(tab_sh.at[ibuf.at[k, 0]], rows.at[b], gsem.at[b])

    def wait_g(b):
        pltpu.make_async_copy(tab_sh.at[iba.at[0, 0]], rows.at[b],
                              gsem.at[b]).wait()

    def fire_s(ibuf, k, b):
        pltpu.async_copy(rows.at[b], S_sh.at[ibuf.at[k, 1]], rsem.at[b],
                         add=True)

    def wait_s(b):
        pltpu.make_async_copy(rows.at[b], S_sh.at[iba.at[0, 1]],
                              rsem.at[b]).wait()

    def do_block(ibuf, notfirst):
        def guarded(f):
            if notfirst is True:
                f()
            else:
                pl.when(notfirst)(f)

        guarded(lambda: wait_s(0))
        fire_g(ibuf, 0, 0)
        for j in range(BLK):
            b = j % NB
            if j < BLK - 1:
                bn = (j + 1) % NB
                if j < NB - 1:
                    guarded(lambda bn=bn: wait_s(bn))
                else:
                    wait_s(bn)
                fire_g(ibuf, j + 1, bn)
            wait_g(b)
            fire_s(ibuf, j, b)

    def load_blk(ibuf, bi):
        pltpu.sync_copy(idx2_hbm.at[pl.ds(c0 + bi * BLK, BLK)], ibuf)

    load_blk(iba, 0)

    @pl.loop(0, P2_NBLK // 2)
    def _(p):
        do_block(iba, p > 0)
        load_blk(ibb, 2 * p + 1)
        do_block(ibb, True)

        @pl.when(p < P2_NBLK // 2 - 1)
        def _():
            load_blk(iba, 2 * p + 2)

    for b in range(NB):
        wait_s(b)

    @pl.loop(0, P2_TAIL)
    def _(k):
        pltpu.sync_copy(idx2_hbm.at[pl.ds(c0 + P2_NBLK * BLK + k, 1)],
                        iba.at[pl.ds(0, 1)])
        pltpu.sync_copy(tab_sh.at[iba.at[0, 0]], rows.at[0])
        pltpu.sync_copy(rows.at[0], S_sh.at[iba.at[0, 1]], add=True)

    @pl.when(t < P2_PER_CORE - NS * P2_PER_TILE)
    def _():
        pltpu.sync_copy(
            idx2_hbm.at[pl.ds(c * P2_PER_CORE + NS * P2_PER_TILE + t, 1)],
            iba.at[pl.ds(0, 1)])
        pltpu.sync_copy(tab_sh.at[iba.at[0, 0]], rows.at[0])
        pltpu.sync_copy(rows.at[0], S_sh.at[iba.at[0, 1]], add=True)

    plsc.subcore_barrier()
    pltpu.sync_copy(S_sh.at[pl.ds(base, ROWS_PER_TILE)],
                    S2_hbm.at[c, pl.ds(base, ROWS_PER_TILE)])


def _sparse2(T, idx2):
    mesh = plsc.VectorSubcoreMesh(core_axis_name="c", subcore_axis_name="s")
    f = pl.kernel(
        _sparse2_body,
        out_type=jax.ShapeDtypeStruct((2, NP, GP), _F32),
        mesh=mesh,
        compiler_params=_SC_PARAMS,
        scratch_types=[
            pltpu.VMEM((BLK, 2, CH), jnp.int32),
            pltpu.VMEM((BLK, 2, CH), jnp.int32),
            pltpu.VMEM((NB, CH, GP), _F32),
            pltpu.VMEM((CH, GP), _F32),
            pltpu.SemaphoreType.DMA((NB,)),
            pltpu.SemaphoreType.DMA((NB,)),
            pltpu.VMEM_SHARED((NP, GP), _F32),
            pltpu.VMEM_SHARED((NP, GP), _F32),
        ],
    )
    return f(T, idx2)


def kernel(x_user, x_movie, edge_index_rates, edge_index_rev,
           W_user, b_user, W_movie, b_movie,
           W1l_um, b1l_um, W1r_um, W1l_mu, b1l_mu, W1r_mu,
           W2l_um, b2l_um, W2r_um, W2l_mu, b2l_mu, W2r_mu):
    idx2 = jnp.stack([edge_index_rates[0].astype(jnp.int32).reshape(NCHUNK, CH),
                      edge_index_rates[1].astype(jnp.int32).reshape(NCHUNK, CH)],
                     axis=1)

    bu = b_user.reshape(1, H)
    bm = b_movie.reshape(1, H)
    b1um = b1l_um.reshape(1, H)
    b1mu = b1l_mu.reshape(1, H)
    W2l = jnp.zeros((H, GP), _F32).at[:, :3].set(W2l_um)
    W2r = jnp.zeros((H, GP), _F32).at[:, :3].set(W2r_um)
    b2l = jnp.zeros((1, GP), _F32).at[0, :3].set(b2l_um)

    Wu128, bu128, Wm128, bm128 = _wcomb(W_user, bu, W_movie, bm,
                                        W1l_um, W1r_mu, W1l_mu, W1r_um)
    U, M, A, B = _encode(x_user, x_movie, Wu128, bu128, Wm128, bm128)
    SS, cm, cu = _sparse1(A, B, idx2)
    cm = cm.reshape(NP, 1)
    cu = cu.reshape(NP, 1)
    T, base = _mid(SS, cm, cu, U, M, b1um, b1mu, W2l, W2r, b2l)
    S2 = _sparse2(T, idx2)
    out16 = _final(S2[0], S2[1], cm, base)
    return out16[:N, :3]


# speedup vs baseline: 1.0830x; 1.0209x over previous
import jax
import jax.numpy as jnp
from jax import lax
from jax.experimental import pallas as pl
from jax.experimental.pallas import tpu as pltpu
from jax.experimental.pallas import tpu_sc as plsc

N = 10000
NP = 10240
E = 320000
D_IN = 128
H = 64
GP = 16

R = 2048
CH = 128
NCHUNK = E // CH
NS = 16
ROWS_PER_TILE = NP // NS

P1_PER_TILE = NCHUNK // NS
BLK = 12
P1_NBLK = P1_PER_TILE // BLK
P2_PER_CORE = NCHUNK // 2
P2_PER_TILE = P2_PER_CORE // NS
P2_NBLK = P2_PER_TILE // BLK
P2_TAIL = P2_PER_TILE - P2_NBLK * BLK
NB = 4

_F32 = jnp.float32
_HI = jax.lax.Precision.HIGHEST
_SC_PARAMS = pltpu.CompilerParams(use_tc_tiling_on_sc=False,
                                  needs_layout_passes=False)


def _wcomb_body(Wu, bu, Wm, bm, W1lum, W1rmu, W1lmu, W1rum,
                Wu128, bu128, Wm128, bm128):
    Wu128[:, 0:H] = jnp.dot(Wu[...], W1lum[...], precision=_HI,
                            preferred_element_type=_F32)
    Wu128[:, H:2 * H] = jnp.dot(Wu[...], W1rmu[...], precision=_HI,
                                preferred_element_type=_F32)
    bu128[:, 0:H] = jnp.dot(bu[...], W1lum[...], precision=_HI,
                            preferred_element_type=_F32)
    bu128[:, H:2 * H] = jnp.dot(bu[...], W1rmu[...], precision=_HI,
                                preferred_element_type=_F32)
    Wm128[:, 0:H] = jnp.dot(Wm[...], W1lmu[...], precision=_HI,
                            preferred_element_type=_F32)
    Wm128[:, H:2 * H] = jnp.dot(Wm[...], W1rum[...], precision=_HI,
                                preferred_element_type=_F32)
    bm128[:, 0:H] = jnp.dot(bm[...], W1lmu[...], precision=_HI,
                            preferred_element_type=_F32)
    bm128[:, H:2 * H] = jnp.dot(bm[...], W1rum[...], precision=_HI,
                                preferred_element_type=_F32)


def _wcomb(Wu, bu, Wm, bm, W1lum, W1rmu, W1lmu, W1rum):
    return pl.pallas_call(
        _wcomb_body,
        out_shape=[jax.ShapeDtypeStruct((D_IN, 2 * H), _F32),
                   jax.ShapeDtypeStruct((1, 2 * H), _F32),
                   jax.ShapeDtypeStruct((D_IN, 2 * H), _F32),
                   jax.ShapeDtypeStruct((1, 2 * H), _F32)],
    )(Wu, bu, Wm, bm, W1lum, W1rmu, W1lmu, W1rum)


def _enc_body(xu, xm, Wu128, bu128, Wm128, bm128, U, M, A, B):
    u = jnp.dot(xu[...], Wu128[...], precision=_HI,
                preferred_element_type=_F32) + bu128[...]
    m = jnp.dot(xm[...], Wm128[...], precision=_HI,
                preferred_element_type=_F32) + bm128[...]
    U[...] = u
    M[...] = m
    A[...] = u[:, 0:H]
    B[...] = m[:, 0:H]


def _encode(xu, xm, Wu128, bu128, Wm128, bm128):
    blk_x = pl.BlockSpec((R, D_IN), lambda i: (i, 0))
    blk_w = pl.BlockSpec((D_IN, 2 * H), lambda i: (0, 0))
    blk_b = pl.BlockSpec((1, 2 * H), lambda i: (0, 0))
    blk_o = pl.BlockSpec((R, 2 * H), lambda i: (i, 0))
    blk_h = pl.BlockSpec((R, H), lambda i: (i, 0))
    out = jax.ShapeDtypeStruct((NP, 2 * H), _F32)
    outh = jax.ShapeDtypeStruct((NP, H), _F32)
    return pl.pallas_call(
        _enc_body,
        grid=(NP // R,),
        in_specs=[blk_x, blk_x, blk_w, blk_b, blk_w, blk_b],
        out_specs=[blk_o, blk_o, blk_h, blk_h],
        out_shape=[out, out, outh, outh],
    )(xu, xm, Wu128, bu128, Wm128, bm128)


def _mid_body(SS, cm, cu, U, M, b1um, b1mu, W2l, W2r, b2l,
              t_out, base_out):
    m1 = jnp.maximum(SS[...][:, 0:H] / jnp.clip(cm[...], 1.0, None)
                     + b1um[...] + M[...][:, H:2 * H], 0.0)
    u1 = jnp.maximum(SS[...][:, H:2 * H] / jnp.clip(cu[...], 1.0, None)
                     + b1mu[...] + U[...][:, H:2 * H], 0.0)
    t_out[...] = jnp.dot(u1, W2l[...], precision=_HI, preferred_element_type=_F32)
    base_out[...] = jnp.dot(m1, W2r[...], precision=_HI,
                            preferred_element_type=_F32) + b2l[...]


def _mid(SS, cm, cu, U, M, b1um, b1mu, W2l, W2r, b2l):
    blk_s = pl.BlockSpec((R, 2 * H), lambda i: (i, 0))
    blk_c = pl.BlockSpec((R, 1), lambda i: (i, 0))
    blk_b = pl.BlockSpec((1, H), lambda i: (0, 0))
    blk_w = pl.BlockSpec((H, GP), lambda i: (0, 0))
    blk_b2 = pl.BlockSpec((1, GP), lambda i: (0, 0))
    blk_o = pl.BlockSpec((R, GP), lambda i: (i, 0))
    out = jax.ShapeDtypeStruct((NP, GP), _F32)
    return pl.pallas_call(
        _mid_body,
        grid=(NP // R,),
        in_specs=[blk_s, blk_c, blk_c, blk_s, blk_s, blk_b, blk_b,
                  blk_w, blk_w, blk_b2],
        out_specs=[blk_o, blk_o],
        out_shape=[out, out],
    )(SS, cm, cu, U, M, b1um, b1mu, W2l, W2r, b2l)


def _final_body(S2a, S2b, cm, base, out):
    out[...] = (S2a[...] + S2b[...]) / jnp.clip(cm[...], 1.0, None) + base[...]


def _final(S2a, S2b, cm, base):
    blk = pl.BlockSpec((R, GP), lambda i: (i, 0))
    blk_c = pl.BlockSpec((R, 1), lambda i: (i, 0))
    return pl.pallas_call(
        _final_body,
        grid=(NP // R,),
        in_specs=[blk, blk, blk_c, blk],
        out_specs=blk,
        out_shape=jax.ShapeDtypeStruct((NP, GP), _F32),
    )(S2a, S2b, cm, base)


def _fill_zeros(ref, nrows, ncols):
    @pl.loop(0, nrows)
    def _(i):
        @pl.loop(0, ncols, step=GP)
        def _(j):
            ref.at[i, pl.ds(j, GP)][...] = jnp.zeros((GP,), _F32)


def _sparse1_body(A_hbm, B_hbm, idx2_hbm,
                  SS_hbm, cm_hbm, cu_hbm,
                  iba, ibb, rows, cnt_v, cbuf, zb64,
                  gsem, rsem, S_sh, c_all):
    c = lax.axis_index("c")
    t = lax.axis_index("s")
    base = t * ROWS_PER_TILE
    c0 = t * P1_PER_TILE
    ones16 = jnp.ones((16,), _F32)

    _fill_zeros(zb64, CH, H)

    @pl.loop(0, NP // 16)
    def _(i):
        cnt_v.at[pl.ds(i * 16, 16)][...] = jnp.zeros((16,), _F32)

    @pl.loop(0, ROWS_PER_TILE // CH)
    def _(k):
        pltpu.sync_copy(zb64, S_sh.at[pl.ds(base + k * CH, CH)])

    plsc.subcore_barrier()

    def run_dir(tab_hbm, gslot, sslot):
        def fire_g(ibuf, k, b):
            pltpu.async_copy(tab_hbm.at[ibuf.at[k, gslot]], rows.at[b],
                             gsem.at[b])

        def wait_g(b):
            pltpu.make_async_copy(tab_hbm.at[iba.at[0, gslot]], rows.at[b],
                                  gsem.at[b]).wait()

        def fire_s(ibuf, k, b):
            pltpu.async_copy(rows.at[b], S_sh.at[ibuf.at[k, sslot]],
                             rsem.at[b], add=True)
            for r in range(CH // 16):
                ireg = ibuf.at[k, sslot, pl.ds(16 * r, 16)][...]
                plsc.addupdate_scatter(cnt_v, [ireg], ones16)

        def wait_s(b):
            pltpu.make_async_copy(rows.at[b], S_sh.at[iba.at[0, sslot]],
                                  rsem.at[b]).wait()

        def do_block(ibuf, notfirst):
            def guarded(f):
                if notfirst is True:
                    f()
                else:
                    pl.when(notfirst)(f)

            guarded(lambda: wait_s(0))
            fire_g(ibuf, 0, 0)
            for j in range(BLK):
                b = j % NB
                if j < BLK - 1:
                    bn = (j + 1) % NB
                    if j < NB - 1:
                        guarded(lambda bn=bn: wait_s(bn))
                    else:
                        wait_s(bn)
                    fire_g(ibuf, j + 1, bn)
                wait_g(b)
                fire_s(ibuf, j, b)

        def load_blk(ibuf, bi):
            pltpu.sync_copy(idx2_hbm.at[pl.ds(c0 + bi * BLK, BLK)], ibuf)

        load_blk(iba, 0)

        @pl.loop(0, P1_NBLK // 2)
        def _(p):
            do_block(iba, p > 0)
            load_blk(ibb, 2 * p + 1)
            do_block(ibb, True)
            load_blk(iba, 2 * p + 2)

        do_block(iba, True)

        for b in range(NB):
            wait_s(b)

        @pl.when(t < NCHUNK - NS * P1_PER_TILE)
        def _():
            pltpu.sync_copy(idx2_hbm.at[pl.ds(NS * P1_PER_TILE + t, 1)],
                            iba.at[pl.ds(0, 1)])
            pltpu.sync_copy(tab_hbm.at[iba.at[0, gslot]], rows.at[0])
            pltpu.sync_copy(rows.at[0], S_sh.at[iba.at[0, sslot]], add=True)
            for r in range(CH // 16):
                ireg = iba.at[0, sslot, pl.ds(16 * r, 16)][...]
                plsc.addupdate_scatter(cnt_v, [ireg], ones16)

    @pl.when(c == 0)
    def _():
        run_dir(A_hbm, 0, 1)

    @pl.when(c == 1)
    def _():
        run_dir(B_hbm, 1, 0)

    pltpu.sync_copy(cnt_v, c_all.at[t])
    plsc.subcore_barrier()
    pltpu.sync_copy(c_all.at[:, pl.ds(base, ROWS_PER_TILE)], cbuf)

    @pl.loop(0, ROWS_PER_TILE // 16)
    def _(g):
        acc = cbuf.at[0, pl.ds(g * 16, 16)][...]
        for r in range(1, NS):
            acc = acc + cbuf.at[r, pl.ds(g * 16, 16)][...]
        cbuf.at[0, pl.ds(g * 16, 16)][...] = acc

    pltpu.sync_copy(S_sh.at[pl.ds(base, ROWS_PER_TILE)],
                    SS_hbm.at[pl.ds(base, ROWS_PER_TILE), pl.ds(c * H, H)])

    @pl.when(c == 0)
    def _():
        pltpu.sync_copy(cbuf.at[0], cm_hbm.at[pl.ds(base, ROWS_PER_TILE)])

    @pl.when(c == 1)
    def _():
        pltpu.sync_copy(cbuf.at[0], cu_hbm.at[pl.ds(base, ROWS_PER_TILE)])


def _sparse1(A, B, idx2):
    mesh = plsc.VectorSubcoreMesh(core_axis_name="c", subcore_axis_name="s")
    out = [jax.ShapeDtypeStruct((NP, 2 * H), _F32),
           jax.ShapeDtypeStruct((NP,), _F32),
           jax.ShapeDtypeStruct((NP,), _F32)]
    f = pl.kernel(
        _sparse1_body,
        out_type=out,
        mesh=mesh,
        compiler_params=_SC_PARAMS,
        scratch_types=[
            pltpu.VMEM((BLK, 2, CH), jnp.int32),
            pltpu.VMEM((BLK, 2, CH), jnp.int32),
            pltpu.VMEM((NB, CH, H), _F32),
            pltpu.VMEM((NP,), _F32),
            pltpu.VMEM((NS, ROWS_PER_TILE), _F32),
            pltpu.VMEM((CH, H), _F32),
            pltpu.SemaphoreType.DMA((NB,)),
            pltpu.SemaphoreType.DMA((NB,)),
            pltpu.VMEM_SHARED((NP, H), _F32),
            pltpu.VMEM_SHARED((NS, NP), _F32),
        ],
    )
    return f(A, B, idx2)


def _sparse2_body(T_hbm, idx2_hbm, S2_hbm,
                  iba, ibb, rows, zb16, gsem, rsem, tab_sh, S_sh):
    c = lax.axis_index("c")
    t = lax.axis_index("s")
    base = t * ROWS_PER_TILE
    c0 = c * P2_PER_CORE + t * P2_PER_TILE
    _fill_zeros(zb16, CH, GP)

    pltpu.sync_copy(T_hbm.at[pl.ds(base, ROWS_PER_TILE)],
                    tab_sh.at[pl.ds(base, ROWS_PER_TILE)])

    @pl.loop(0, ROWS_PER_TILE // CH)
    def _(k):
        pltpu.sync_copy(zb16, S_sh.at[pl.ds(base + k * CH, CH)])

    plsc.subcore_barrier()

    def fire_g(ibuf, k, b):
        pltpu.async_copy(tab_sh.at[ibuf.at[k, 0]], rows.at[b], gsem.at[b])

    def wait_g(b):
        pltpu.make_async_copy(tab_sh.at[iba.at[0, 0]], rows.at[b],
                              gsem.at[b]).wait()

    def fire_s(ibuf, k, b):
        pltpu.async_copy(rows.at[b], S_sh.at[ibuf.at[k, 1]], rsem.at[b],
                         add=True)

    def wait_s(b):
        pltpu.make_async_copy(rows.at[b], S_sh.at[iba.at[0, 1]],
                              rsem.at[b]).wait()

    def do_block(ibuf, notfirst):
        def guarded(f):
            if notfirst is True:
                f()
            else:
                pl.when(notfirst)(f)

        guarded(lambda: wait_s(0))
        fire_g(ibuf, 0, 0)
        for j in range(BLK):
            b = j % NB
            if j < BLK - 1:
                bn = (j + 1) % NB
                if j < NB - 1:
                    guarded(lambda bn=bn: wait_s(bn))
                else:
                    wait_s(bn)
                fire_g(ibuf, j + 1, bn)
            wait_g(b)
            fire_s(ibuf, j, b)

    def load_blk(ibuf, bi):
        pltpu.sync_copy(idx2_hbm.at[pl.ds(c0 + bi * BLK, BLK)], ibuf)

    load_blk(iba, 0)

    @pl.loop(0, P2_NBLK // 2)
    def _(p):
        do_block(iba, p > 0)
        load_blk(ibb, 2 * p + 1)
        do_block(ibb, True)

        @pl.when(p < P2_NBLK // 2 - 1)
        def _():
            load_blk(iba, 2 * p + 2)

    for b in range(NB):
        wait_s(b)

    @pl.loop(0, P2_TAIL)
    def _(k):
        pltpu.sync_copy(idx2_hbm.at[pl.ds(c0 + P2_NBLK * BLK + k, 1)],
                        iba.at[pl.ds(0, 1)])
        pltpu.sync_copy(tab_sh.at[iba.at[0, 0]], rows.at[0])
        pltpu.sync_copy(rows.at[0], S_sh.at[iba.at[0, 1]], add=True)

    @pl.when(t < P2_PER_CORE - NS * P2_PER_TILE)
    def _():
        pltpu.sync_copy(
            idx2_hbm.at[pl.ds(c * P2_PER_CORE + NS * P2_PER_TILE + t, 1)],
            iba.at[pl.ds(0, 1)])
        pltpu.sync_copy(tab_sh.at[iba.at[0, 0]], rows.at[0])
        pltpu.sync_copy(rows.at[0], S_sh.at[iba.at[0, 1]], add=True)

    plsc.subcore_barrier()
    pltpu.sync_copy(S_sh.at[pl.ds(base, ROWS_PER_TILE)],
                    S2_hbm.at[c, pl.ds(base, ROWS_PER_TILE)])


def _sparse2(T, idx2):
    mesh = plsc.VectorSubcoreMesh(core_axis_name="c", subcore_axis_name="s")
    f = pl.kernel(
        _sparse2_body,
        out_type=jax.ShapeDtypeStruct((2, NP, GP), _F32),
        mesh=mesh,
        compiler_params=_SC_PARAMS,
        scratch_types=[
            pltpu.VMEM((BLK, 2, CH), jnp.int32),
            pltpu.VMEM((BLK, 2, CH), jnp.int32),
            pltpu.VMEM((NB, CH, GP), _F32),
            pltpu.VMEM((CH, GP), _F32),
            pltpu.SemaphoreType.DMA((NB,)),
            pltpu.SemaphoreType.DMA((NB,)),
            pltpu.VMEM_SHARED((NP, GP), _F32),
            pltpu.VMEM_SHARED((NP, GP), _F32),
        ],
    )
    return f(T, idx2)


def kernel(x_user, x_movie, edge_index_rates, edge_index_rev,
           W_user, b_user, W_movie, b_movie,
           W1l_um, b1l_um, W1r_um, W1l_mu, b1l_mu, W1r_mu,
           W2l_um, b2l_um, W2r_um, W2l_mu, b2l_mu, W2r_mu):
    idx2 = jnp.stack([edge_index_rates[0].astype(jnp.int32).reshape(NCHUNK, CH),
                      edge_index_rates[1].astype(jnp.int32).reshape(NCHUNK, CH)],
                     axis=1)

    bu = b_user.reshape(1, H)
    bm = b_movie.reshape(1, H)
    b1um = b1l_um.reshape(1, H)
    b1mu = b1l_mu.reshape(1, H)
    W2l = jnp.zeros((H, GP), _F32).at[:, :3].set(W2l_um)
    W2r = jnp.zeros((H, GP), _F32).at[:, :3].set(W2r_um)
    b2l = jnp.zeros((1, GP), _F32).at[0, :3].set(b2l_um)

    Wu128, bu128, Wm128, bm128 = _wcomb(W_user, bu, W_movie, bm,
                                        W1l_um, W1r_mu, W1l_mu, W1r_um)
    U, M, A, B = _encode(x_user, x_movie, Wu128, bu128, Wm128, bm128)
    SS, cm, cu = _sparse1(A, B, idx2)
    cm = cm.reshape(NP, 1)
    cu = cu.reshape(NP, 1)
    T, base = _mid(SS, cm, cu, U, M, b1um, b1mu, W2l, W2r, b2l)
    S2 = _sparse2(T, idx2)
    out16 = _final(S2[0], S2[1], cm, base)
    return out16[:N, :3]
